# baseline (speedup 1.0000x reference)
import functools

import jax
import jax.numpy as jnp
from jax.experimental import pallas as pl

N_NODES = 10000
D_IN = 256
D_HID = 512
D_OUT = 256
ROW_BLK = 1000


def _layer1_body(a_ref, x_ref, deg_ref, wl_ref, wr_ref, b_ref, o_ref):
    inv = 1.0 / jnp.maximum(deg_ref[...], 1.0)
    a = jnp.concatenate([a_ref[0], a_ref[1]], axis=1) * inv
    h = jnp.dot(a, wl_ref[...], preferred_element_type=jnp.float32)
    h += jnp.dot(x_ref[...], wr_ref[...], preferred_element_type=jnp.float32)
    h = jnp.maximum(h + b_ref[...], 0.0)
    for j in range(4):
        o_ref[j] = h[:, j * 128:(j + 1) * 128]


def _layer2_body(a_ref, h_ref, deg_ref, wl_ref, wr_ref, b_ref,
                 wlin_ref, blin_ref, o_ref):
    inv = 1.0 / jnp.maximum(deg_ref[...], 1.0)
    a = jnp.concatenate([a_ref[j] for j in range(4)], axis=1) * inv
    hprev = jnp.concatenate([h_ref[j] for j in range(4)], axis=1)
    h = jnp.dot(a, wl_ref[...], preferred_element_type=jnp.float32)
    h += jnp.dot(hprev, wr_ref[...], preferred_element_type=jnp.float32)
    h = jnp.maximum(h + b_ref[...], 0.0)
    o_ref[...] = jnp.dot(h, wlin_ref[...], preferred_element_type=jnp.float32)
    o_ref[...] += blin_ref[...]


def _tc_layer1(agg1, x, deg, Wl1, Wr1, b1):
    grid = (N_NODES // ROW_BLK,)
    return pl.pallas_call(
        _layer1_body,
        grid=grid,
        in_specs=[
            pl.BlockSpec((2, ROW_BLK, 128), lambda i: (0, i, 0)),
            pl.BlockSpec((ROW_BLK, D_IN), lambda i: (i, 0)),
            pl.BlockSpec((ROW_BLK, 1), lambda i: (i, 0)),
            pl.BlockSpec((D_IN, D_HID), lambda i: (0, 0)),
            pl.BlockSpec((D_IN, D_HID), lambda i: (0, 0)),
            pl.BlockSpec((1, D_HID), lambda i: (0, 0)),
        ],
        out_specs=pl.BlockSpec((4, ROW_BLK, 128), lambda i: (0, i, 0)),
        out_shape=jax.ShapeDtypeStruct((4, N_NODES, 128), jnp.float32),
    )(agg1, x, deg, Wl1, Wr1, b1)


def _tc_layer2(agg2, h1, deg, Wl2, Wr2, b2, Wlin, blin):
    grid = (N_NODES // ROW_BLK,)
    return pl.pallas_call(
        _layer2_body,
        grid=grid,
        in_specs=[
            pl.BlockSpec((4, ROW_BLK, 128), lambda i: (0, i, 0)),
            pl.BlockSpec((4, ROW_BLK, 128), lambda i: (0, i, 0)),
            pl.BlockSpec((ROW_BLK, 1), lambda i: (i, 0)),
            pl.BlockSpec((D_HID, D_HID), lambda i: (0, 0)),
            pl.BlockSpec((D_HID, D_HID), lambda i: (0, 0)),
            pl.BlockSpec((1, D_HID), lambda i: (0, 0)),
            pl.BlockSpec((D_HID, D_OUT), lambda i: (0, 0)),
            pl.BlockSpec((1, D_OUT), lambda i: (0, 0)),
        ],
        out_specs=pl.BlockSpec((ROW_BLK, D_OUT), lambda i: (i, 0)),
        out_shape=jax.ShapeDtypeStruct((N_NODES, D_OUT), jnp.float32),
    )(agg2, h1, deg, Wl2, Wr2, b2, Wlin, blin)


def _segsum_chunked(table, src, dst, nchunks):
    outs = []
    for c in range(nchunks):
        msg = jnp.take(table, src + c * N_NODES, axis=0)
        outs.append(jax.ops.segment_sum(msg, dst, num_segments=N_NODES))
    return jnp.stack(outs, axis=0)


def kernel(x, edge_index, Wl1, Wr1, b1, Wl2, Wr2, b2, Wlin, blin):
    src = edge_index[0].astype(jnp.int32)
    dst = edge_index[1].astype(jnp.int32)

    deg = jax.ops.segment_sum(jnp.ones((src.shape[0],), jnp.float32), dst,
                              num_segments=N_NODES)[:, None]

    x_t = x.reshape(N_NODES, 2, 128).transpose(1, 0, 2).reshape(2 * N_NODES, 128)
    agg1 = _segsum_chunked(x_t, src, dst, 2)

    h1 = _tc_layer1(agg1, x, deg, Wl1, Wr1, b1.reshape(1, D_HID))

    h1_t = h1.reshape(4 * N_NODES, 128)
    agg2 = _segsum_chunked(h1_t, src, dst, 4)

    return _tc_layer2(agg2, h1, deg, Wl2, Wr2, b2.reshape(1, D_HID),
                      Wlin, blin.reshape(1, D_OUT))
